# baseline (speedup 1.0000x reference)
import functools

import jax
import jax.numpy as jnp
from jax import lax
from jax.experimental import pallas as pl
from jax.experimental.pallas import tpu as pltpu
from jax.experimental.pallas import tpu_sc as plsc

B, D, H, E, L, K = 512, 768, 768, 8, 2, 2
F = 4 * D
T = 128
NT = 16
P = NT * T
NC, NS = 2, 16
NW = NC * NS
RPW = P // NW

_DN_T = (((1,), (1,)), ((), ()))


def _mm_t(a, w):
    return lax.dot_general(a, w, _DN_T, preferred_element_type=jnp.float32)


def _mm(a, w):
    return lax.dot_general(a, w, (((1,), (0,)), ((), ())),
                           preferred_element_type=jnp.float32)


def _router_body(x_ref, h_ref, Wp_ref, bp_ref, Wih_ref, Whh_ref, bih_ref,
                 bhh_ref, Wr_ref, br_ref,
                 hn_ref, pos_ref, gates_ref, te_ref, valid_ref, src_ref):
    xf = x_ref[...]
    xp = _mm_t(xf, Wp_ref[...]) + bp_ref[...]
    gi = _mm_t(xp, Wih_ref[...]) + bih_ref[...]
    gh = _mm_t(h_ref[...], Whh_ref[...]) + bhh_ref[...]
    i_r, i_z, i_n = gi[:, :H], gi[:, H:2 * H], gi[:, 2 * H:]
    h_r, h_z, h_n = gh[:, :H], gh[:, H:2 * H], gh[:, 2 * H:]
    r = jax.nn.sigmoid(i_r + h_r)
    z = jax.nn.sigmoid(i_z + h_z)
    n = jnp.tanh(i_n + r * h_n)
    hn = (1.0 - z) * n + z * h_ref[...]
    hn_ref[...] = hn

    logits = _mm_t(hn, Wr_ref[...]) + br_ref[...]
    m = jnp.max(logits, axis=-1, keepdims=True)
    p = jnp.exp(logits - m)
    p = p / jnp.sum(p, axis=-1, keepdims=True)
    ii = lax.broadcasted_iota(jnp.int32, (B, E), 1)
    m1 = jnp.max(p, axis=-1, keepdims=True)
    i1 = jnp.min(jnp.where(p >= m1, ii, E), axis=-1, keepdims=True)
    p2 = jnp.where(ii == i1, -1e30, p)
    m2 = jnp.max(p2, axis=-1, keepdims=True)
    i2 = jnp.min(jnp.where(p2 >= m2, ii, E), axis=-1, keepdims=True)
    s = m1 + m2
    g1 = m1 / s
    g2 = m2 / s
    gates_ref[...] = jnp.concatenate([g1, g2], axis=1)

    oh1 = (ii == i1).astype(jnp.float32)
    oh2 = (ii == i2).astype(jnp.float32)
    tot = oh1 + oh2

    ri = lax.broadcasted_iota(jnp.int32, (B, B), 0)
    ci = lax.broadcasted_iota(jnp.int32, (B, B), 1)
    lt = (ci < ri).astype(jnp.float32)
    excl = _mm(lt, tot)

    counts = jnp.sum(tot, axis=0, keepdims=True)
    pc = jnp.ceil(counts * (1.0 / T)) * T
    ei = lax.broadcasted_iota(jnp.int32, (E, E), 0)
    ej = lax.broadcasted_iota(jnp.int32, (E, E), 1)
    le = (ei <= ej).astype(jnp.float32)
    cum = _mm(pc, le)
    poff = cum - pc

    base_e = poff + excl
    pos1 = jnp.sum(oh1 * base_e, axis=1, keepdims=True)
    pos2 = jnp.sum(oh2 * base_e, axis=1, keepdims=True)
    pos_ref[...] = jnp.concatenate([pos1, pos2], axis=1).astype(jnp.int32)

    iden = (ei == ej).astype(jnp.float32)
    cum_col = lax.dot_general(iden, cum, (((1,), (1,)), ((), ())),
                              preferred_element_type=jnp.float32)
    bases = (T * lax.broadcasted_iota(jnp.int32, (1, NT), 1)).astype(
        jnp.float32)
    cmp = (jnp.broadcast_to(bases, (E, NT))
           >= jnp.broadcast_to(cum_col, (E, NT))).astype(jnp.float32)
    te_raw = jnp.sum(cmp, axis=0, keepdims=True)
    total = cum[0:1, E - 1:E]
    valid_ref[...] = (bases < total).astype(jnp.int32)
    nvt = (total * (1.0 / T)).astype(jnp.int32)
    rt = lax.broadcasted_iota(jnp.int32, (NT, NT), 0)
    ct = lax.broadcasted_iota(jnp.int32, (NT, NT), 1)
    sel = (rt == jnp.minimum(ct, nvt - 1)).astype(jnp.float32)
    te_ref[...] = _mm(te_raw, sel).astype(jnp.int32)

    lane_p = lax.broadcasted_iota(jnp.int32, (B, P), 1)
    n_col = lax.broadcasted_iota(jnp.int32, (B, P), 0)
    p1i = pos1.astype(jnp.int32)
    p2i = pos2.astype(jnp.int32)
    contrib = (jnp.where(lane_p == p1i, n_col, 0)
               + jnp.where(lane_p == p2i, n_col, 0))
    src_ref[...] = jnp.sum(contrib, axis=0, keepdims=True)


def _router_call(x2, h, Wp_l, bp_l, W_ih, W_hh, b_ih2, b_hh2, Wr_l, br_l,
                 interpret=False):
    out_shapes = (
        jax.ShapeDtypeStruct((B, H), jnp.float32),
        jax.ShapeDtypeStruct((B, K), jnp.int32),
        jax.ShapeDtypeStruct((B, K), jnp.float32),
        jax.ShapeDtypeStruct((1, NT), jnp.int32),
        jax.ShapeDtypeStruct((1, NT), jnp.int32),
        jax.ShapeDtypeStruct((1, P), jnp.int32),
    )
    return pl.pallas_call(
        _router_body,
        out_shape=out_shapes,
        interpret=interpret,
    )(x2, h, Wp_l, bp_l, W_ih, W_hh, b_ih2, b_hh2, Wr_l, br_l)


def _ffn_body(te_ref, valid_ref, xg_ref, W1_ref, b1_ref, W2_ref, b2_ref,
              pos_ref, gates_ref, x2_ref, out_ref, eo_acc):
    t = pl.program_id(0)

    @pl.when(t == 0)
    def _init():
        eo_acc[...] = jnp.zeros_like(eo_acc)

    @pl.when(valid_ref[t] == 1)
    def _tile():
        xg = xg_ref[...]
        h1 = jnp.maximum(_mm_t(xg, W1_ref[0]) + b1_ref[0, 0], 0.0)
        y = _mm_t(h1, W2_ref[0]) + b2_ref[0, 0]
        lane_t = lax.broadcasted_iota(jnp.int32, (B, T), 1) + t * T
        p0 = pos_ref[:, 0:1]
        p1 = pos_ref[:, 1:2]
        m01 = (jnp.where(lane_t == p0, 1.0, 0.0)
               + jnp.where(lane_t == p1, 1.0, 0.0))
        u = _mm(m01, y)
        g0 = gates_ref[:, 0:1]
        g1 = gates_ref[:, 1:2]
        gcol = (jnp.where(p0 // T == t, g0, 0.0)
                + jnp.where(p1 // T == t, g1, 0.0))
        eo_acc[...] += gcol * u

    @pl.when(t == NT - 1)
    def _fin():
        out_ref[...] = x2_ref[...] + eo_acc[...]


def _ffn_call(te, valid, xg, W1_l, b1_l, W2_l, b2_l, pos, gates, x2,
              interpret=False):
    grid_spec = pltpu.PrefetchScalarGridSpec(
        num_scalar_prefetch=2,
        grid=(NT,),
        in_specs=[
            pl.BlockSpec((T, D), lambda t, te, vld: (t, 0)),
            pl.BlockSpec((1, F, D), lambda t, te, vld: (te[t], 0, 0)),
            pl.BlockSpec((1, 1, F), lambda t, te, vld: (te[t], 0, 0)),
            pl.BlockSpec((1, D, F), lambda t, te, vld: (te[t], 0, 0)),
            pl.BlockSpec((1, 1, D), lambda t, te, vld: (te[t], 0, 0)),
            pl.BlockSpec((B, K), lambda t, te, vld: (0, 0)),
            pl.BlockSpec((B, K), lambda t, te, vld: (0, 0)),
            pl.BlockSpec((B, D), lambda t, te, vld: (0, 0)),
        ],
        out_specs=pl.BlockSpec((B, D), lambda t, te, vld: (0, 0)),
        scratch_shapes=[pltpu.VMEM((B, D), jnp.float32)],
    )
    return pl.pallas_call(
        _ffn_body,
        grid_spec=grid_spec,
        out_shape=jax.ShapeDtypeStruct((B, D), jnp.float32),
        compiler_params=pltpu.CompilerParams(
            dimension_semantics=("arbitrary",),
        ),
        interpret=interpret,
    )(te, valid, xg, W1_l, b1_l, W2_l, b2_l, pos, gates, x2)


@functools.cache
def _get_sc_gather():
    @functools.partial(
        pl.kernel,
        out_type=jax.ShapeDtypeStruct((P, D), jnp.float32),
        mesh=plsc.VectorSubcoreMesh(core_axis_name="c", subcore_axis_name="s"),
        scratch_types=[
            pltpu.VMEM((RPW,), jnp.int32),
            pltpu.VMEM((RPW, D), jnp.float32),
            pltpu.SemaphoreType.DMA,
        ],
    )
    def _sc_gather(src_hbm, x_hbm, out_hbm, idx_v, rows_v, sem):
        wid = lax.axis_index("s") * NC + lax.axis_index("c")
        base = wid * RPW
        pltpu.sync_copy(src_hbm.at[pl.ds(base, RPW)], idx_v)
        pltpu.async_copy(x_hbm.at[idx_v], rows_v, sem).wait()
        pltpu.sync_copy(rows_v, out_hbm.at[pl.ds(base, RPW)])

    return _sc_gather


@jax.jit
def _pipeline(x2, Wp, bp, W_ih, W_hh, b_ih2, b_hh2, Wr, br, W1e, b1e, W2e,
              b2e):
    h = jnp.zeros((B, H), jnp.float32)
    for l in range(L):
        hn, pos, gates, te, valid, src = _router_call(
            x2, h, Wp[l], bp[l].reshape(1, H), W_ih, W_hh, b_ih2, b_hh2,
            Wr[l], br[l].reshape(1, E))
        xg = _get_sc_gather()(src.reshape(P), x2)
        x2 = _ffn_call(te.reshape(NT), valid.reshape(NT), xg,
                       W1e[l], b1e[l].reshape(E, 1, F),
                       W2e[l], b2e[l].reshape(E, 1, D), pos, gates, x2)
        h = hn
    return x2


def kernel(x, Wp, bp, W_ih, W_hh, b_ih, b_hh, Wr, br, W1e, b1e, W2e, b2e):
    batch, seq, d = x.shape
    x2 = x.reshape(batch * seq, d)
    out = _pipeline(x2, Wp, bp, W_ih, W_hh, b_ih.reshape(1, 3 * H),
                    b_hh.reshape(1, 3 * H), Wr, br, W1e, b1e, W2e, b2e)
    return out.reshape(batch, seq, d)


# speedup vs baseline: 1.3387x; 1.3387x over previous
import functools

import jax
import jax.numpy as jnp
from jax import lax
from jax.experimental import pallas as pl
from jax.experimental.pallas import tpu as pltpu
from jax.experimental.pallas import tpu_sc as plsc

B, D, H, E, L, K = 512, 768, 768, 8, 2, 2
F = 4 * D
T = 128
NT = 16
P = NT * T
NC, NS = 2, 16
NW = NC * NS
RPW = P // NW

_DN_T = (((1,), (1,)), ((), ()))


def _mm_t(a, w):
    return lax.dot_general(a, w, _DN_T, preferred_element_type=jnp.float32)


def _mm(a, w):
    return lax.dot_general(a, w, (((1,), (0,)), ((), ())),
                           preferred_element_type=jnp.float32)


def _router_body(x_ref, h_ref, Wp_ref, bp_ref, Wih_ref, Whh_ref, bih_ref,
                 bhh_ref, Wr_ref, br_ref,
                 hn_ref, pos_ref, gates_ref, te_ref, valid_ref):
    xf = x_ref[...]
    xp = _mm_t(xf, Wp_ref[...]) + bp_ref[...]
    gi = _mm_t(xp, Wih_ref[...]) + bih_ref[...]
    gh = _mm_t(h_ref[...], Whh_ref[...]) + bhh_ref[...]
    i_r, i_z, i_n = gi[:, :H], gi[:, H:2 * H], gi[:, 2 * H:]
    h_r, h_z, h_n = gh[:, :H], gh[:, H:2 * H], gh[:, 2 * H:]
    r = jax.nn.sigmoid(i_r + h_r)
    z = jax.nn.sigmoid(i_z + h_z)
    n = jnp.tanh(i_n + r * h_n)
    hn = (1.0 - z) * n + z * h_ref[...]
    hn_ref[...] = hn

    logits = _mm_t(hn, Wr_ref[...]) + br_ref[...]
    m = jnp.max(logits, axis=-1, keepdims=True)
    p = jnp.exp(logits - m)
    p = p / jnp.sum(p, axis=-1, keepdims=True)
    ii = lax.broadcasted_iota(jnp.int32, (B, E), 1)
    m1 = jnp.max(p, axis=-1, keepdims=True)
    i1 = jnp.min(jnp.where(p >= m1, ii, E), axis=-1, keepdims=True)
    p2 = jnp.where(ii == i1, -1e30, p)
    m2 = jnp.max(p2, axis=-1, keepdims=True)
    i2 = jnp.min(jnp.where(p2 >= m2, ii, E), axis=-1, keepdims=True)
    s = m1 + m2
    g1 = m1 / s
    g2 = m2 / s
    gates_ref[...] = jnp.concatenate([g1, g2], axis=1)

    oh1 = (ii == i1).astype(jnp.float32)
    oh2 = (ii == i2).astype(jnp.float32)
    tot = oh1 + oh2

    ri = lax.broadcasted_iota(jnp.int32, (B, B), 0)
    ci = lax.broadcasted_iota(jnp.int32, (B, B), 1)
    lt = (ci < ri).astype(jnp.float32)
    excl = _mm(lt, tot)

    counts = jnp.sum(tot, axis=0, keepdims=True)
    pc = jnp.ceil(counts * (1.0 / T)) * T
    ei = lax.broadcasted_iota(jnp.int32, (E, E), 0)
    ej = lax.broadcasted_iota(jnp.int32, (E, E), 1)
    le = (ei <= ej).astype(jnp.float32)
    cum = _mm(pc, le)
    poff = cum - pc

    base_e = poff + excl
    pos1 = jnp.sum(oh1 * base_e, axis=1, keepdims=True)
    pos2 = jnp.sum(oh2 * base_e, axis=1, keepdims=True)
    pos_ref[...] = jnp.concatenate([pos1, pos2], axis=1).astype(jnp.int32)

    iden = (ei == ej).astype(jnp.float32)
    cum_col = lax.dot_general(iden, cum, (((1,), (1,)), ((), ())),
                              preferred_element_type=jnp.float32)
    bases = (T * lax.broadcasted_iota(jnp.int32, (1, NT), 1)).astype(
        jnp.float32)
    cmp = (jnp.broadcast_to(bases, (E, NT))
           >= jnp.broadcast_to(cum_col, (E, NT))).astype(jnp.float32)
    te_raw = jnp.sum(cmp, axis=0, keepdims=True)
    total = cum[0:1, E - 1:E]
    valid_ref[...] = (bases < total).astype(jnp.int32)
    nvt = (total * (1.0 / T)).astype(jnp.int32)
    rt = lax.broadcasted_iota(jnp.int32, (NT, NT), 0)
    ct = lax.broadcasted_iota(jnp.int32, (NT, NT), 1)
    sel = (rt == jnp.minimum(ct, nvt - 1)).astype(jnp.float32)
    te_ref[...] = _mm(te_raw, sel).astype(jnp.int32)

def _router_call(x2, h, Wp_l, bp_l, W_ih, W_hh, b_ih2, b_hh2, Wr_l, br_l,
                 interpret=False):
    out_shapes = (
        jax.ShapeDtypeStruct((B, H), jnp.float32),
        jax.ShapeDtypeStruct((B, K), jnp.int32),
        jax.ShapeDtypeStruct((B, K), jnp.float32),
        jax.ShapeDtypeStruct((1, NT), jnp.int32),
        jax.ShapeDtypeStruct((1, NT), jnp.int32),
    )
    return pl.pallas_call(
        _router_body,
        out_shape=out_shapes,
        interpret=interpret,
    )(x2, h, Wp_l, bp_l, W_ih, W_hh, b_ih2, b_hh2, Wr_l, br_l)


def _ffn_body(te_ref, valid_ref, W1_ref, b1_ref, W2_ref, b2_ref,
              pos_ref, gates_ref, x2_ref, out_ref, eo_acc):
    t = pl.program_id(0)

    @pl.when(t == 0)
    def _init():
        eo_acc[...] = jnp.zeros_like(eo_acc)

    @pl.when(valid_ref[t] == 1)
    def _tile():
        lane_t = lax.broadcasted_iota(jnp.int32, (B, T), 1) + t * T
        p0 = pos_ref[:, 0:1]
        p1 = pos_ref[:, 1:2]
        m01 = (jnp.where(lane_t == p0, 1.0, 0.0)
               + jnp.where(lane_t == p1, 1.0, 0.0))
        xg = lax.dot_general(m01, x2_ref[...], (((0,), (0,)), ((), ())),
                             preferred_element_type=jnp.float32)
        h1 = jnp.maximum(_mm_t(xg, W1_ref[0]) + b1_ref[0, 0], 0.0)
        y = _mm_t(h1, W2_ref[0]) + b2_ref[0, 0]
        u = _mm(m01, y)
        g0 = gates_ref[:, 0:1]
        g1 = gates_ref[:, 1:2]
        gcol = (jnp.where(p0 // T == t, g0, 0.0)
                + jnp.where(p1 // T == t, g1, 0.0))
        eo_acc[...] += gcol * u

    @pl.when(t == NT - 1)
    def _fin():
        out_ref[...] = x2_ref[...] + eo_acc[...]


def _ffn_call(te, valid, W1_l, b1_l, W2_l, b2_l, pos, gates, x2,
              interpret=False):
    grid_spec = pltpu.PrefetchScalarGridSpec(
        num_scalar_prefetch=2,
        grid=(NT,),
        in_specs=[
            pl.BlockSpec((1, F, D), lambda t, te, vld: (te[t], 0, 0)),
            pl.BlockSpec((1, 1, F), lambda t, te, vld: (te[t], 0, 0)),
            pl.BlockSpec((1, D, F), lambda t, te, vld: (te[t], 0, 0)),
            pl.BlockSpec((1, 1, D), lambda t, te, vld: (te[t], 0, 0)),
            pl.BlockSpec((B, K), lambda t, te, vld: (0, 0)),
            pl.BlockSpec((B, K), lambda t, te, vld: (0, 0)),
            pl.BlockSpec((B, D), lambda t, te, vld: (0, 0)),
        ],
        out_specs=pl.BlockSpec((B, D), lambda t, te, vld: (0, 0)),
        scratch_shapes=[pltpu.VMEM((B, D), jnp.float32)],
    )
    return pl.pallas_call(
        _ffn_body,
        grid_spec=grid_spec,
        out_shape=jax.ShapeDtypeStruct((B, D), jnp.float32),
        compiler_params=pltpu.CompilerParams(
            dimension_semantics=("arbitrary",),
        ),
        interpret=interpret,
    )(te, valid, W1_l, b1_l, W2_l, b2_l, pos, gates, x2)


@jax.jit
def _pipeline(x2, Wp, bp, W_ih, W_hh, b_ih2, b_hh2, Wr, br, W1e, b1e, W2e,
              b2e):
    h = jnp.zeros((B, H), jnp.float32)
    for l in range(L):
        hn, pos, gates, te, valid = _router_call(
            x2, h, Wp[l], bp[l].reshape(1, H), W_ih, W_hh, b_ih2, b_hh2,
            Wr[l], br[l].reshape(1, E))
        x2 = _ffn_call(te.reshape(NT), valid.reshape(NT),
                       W1e[l], b1e[l].reshape(E, 1, F),
                       W2e[l], b2e[l].reshape(E, 1, D), pos, gates, x2)
        h = hn
    return x2


def kernel(x, Wp, bp, W_ih, W_hh, b_ih, b_hh, Wr, br, W1e, b1e, W2e, b2e):
    batch, seq, d = x.shape
    x2 = x.reshape(batch * seq, d)
    out = _pipeline(x2, Wp, bp, W_ih, W_hh, b_ih.reshape(1, 3 * H),
                    b_hh.reshape(1, 3 * H), Wr, br, W1e, b1e, W2e, b2e)
    return out.reshape(batch, seq, d)


# speedup vs baseline: 3.1488x; 2.3522x over previous
import functools

import jax
import jax.numpy as jnp
from jax import lax
from jax.experimental import pallas as pl
from jax.experimental.pallas import tpu as pltpu

B, D, H, E, L, K = 512, 768, 768, 8, 2, 2
F = 4 * D
NF = 2
FB = F // NF

_DN_T = (((1,), (1,)), ((), ()))


def _mm_t(a, w):
    return lax.dot_general(a, w, _DN_T, preferred_element_type=jnp.float32)


def _chain3(h1, w2, base):
    acc = _mm_t(h1[:, base:base + 256], w2[:, base:base + 256])
    acc = acc + _mm_t(h1[:, base + 256:base + 512],
                      w2[:, base + 256:base + 512])
    return acc + _mm_t(h1[:, base + 512:base + 768],
                       w2[:, base + 512:base + 768])


def _dense_body(x_ref, Wp_ref, bp_ref, Wih_ref, Whh_ref, bih_ref, bhh_ref,
                Wr_ref, br_ref, W1_ref, b1_ref, W2_ref, b2_ref,
                out_ref, xcur, h, wcomb, eo, o_acc):
    l = pl.program_id(0)
    e = pl.program_id(1)
    f = pl.program_id(2)

    @pl.when(jnp.logical_and(l == 0, jnp.logical_and(e == 0, f == 0)))
    def _init():
        xcur[...] = x_ref[...]
        h[...] = jnp.zeros_like(h)

    @pl.when(jnp.logical_and(e == 0, f == 0))
    def _router():
        xf = xcur[...]
        xp = _mm_t(xf, Wp_ref[0]) + bp_ref[0]
        gi = _mm_t(xp, Wih_ref[...]) + bih_ref[0]
        gh = _mm_t(h[...], Whh_ref[...]) + bhh_ref[0]
        i_r, i_z, i_n = gi[:, :H], gi[:, H:2 * H], gi[:, 2 * H:]
        h_r, h_z, h_n = gh[:, :H], gh[:, H:2 * H], gh[:, 2 * H:]
        r = jax.nn.sigmoid(i_r + h_r)
        z = jax.nn.sigmoid(i_z + h_z)
        n = jnp.tanh(i_n + r * h_n)
        hn = (1.0 - z) * n + z * h[...]
        h[...] = hn

        logits = _mm_t(hn, Wr_ref[0]) + br_ref[0]
        m = jnp.max(logits, axis=-1, keepdims=True)
        p = jnp.exp(logits - m)
        p = p / jnp.sum(p, axis=-1, keepdims=True)
        ii = lax.broadcasted_iota(jnp.int32, (B, E), 1)
        m1 = jnp.max(p, axis=-1, keepdims=True)
        i1 = jnp.min(jnp.where(p >= m1, ii, E), axis=-1, keepdims=True)
        p2 = jnp.where(ii == i1, -1e30, p)
        m2 = jnp.max(p2, axis=-1, keepdims=True)
        i2 = jnp.min(jnp.where(p2 >= m2, ii, E), axis=-1, keepdims=True)
        s = m1 + m2
        wcomb[...] = (jnp.where(ii == i1, m1 / s, 0.0)
                      + jnp.where(ii == i2, m2 / s, 0.0))
        eo[...] = jnp.zeros_like(eo)

    xf = xcur[...]
    h1 = jnp.maximum(_mm_t(xf, W1_ref[0, 0]) + b1_ref[0, 0], 0.0)
    w2 = W2_ref[0, 0]
    ca = _chain3(h1, w2, 0)
    cb = _chain3(h1, w2, 768)

    @pl.when(f == 0)
    def _half0():
        o_acc[...] = ca + cb

    @pl.when(f == 1)
    def _half1():
        y = ((o_acc[...] + ca) + cb) + b2_ref[0, 0]
        ii2 = lax.broadcasted_iota(jnp.int32, (B, E), 1)
        gate = jnp.sum(jnp.where(ii2 == e, wcomb[...], 0.0), axis=-1,
                       keepdims=True)
        eo[...] = eo[...] + gate * y

    @pl.when(jnp.logical_and(e == E - 1, f == NF - 1))
    def _fin():
        xn = xf + eo[...]
        xcur[...] = xn

        @pl.when(l == L - 1)
        def _out():
            out_ref[...] = xn


@functools.partial(jax.jit, static_argnames=("interpret",))
def _run(x2, Wp, bp3, W_ih, W_hh, b_ih2, b_hh2, Wr, br3, W1e, b1e4, W2e, b2e4,
         interpret=False):
    grid = (L, E, NF)
    specs = [
        pl.BlockSpec((B, D), lambda l, e, f: (0, 0)),
        pl.BlockSpec((1, H, D), lambda l, e, f: (l, 0, 0)),
        pl.BlockSpec((1, 1, H), lambda l, e, f: (l, 0, 0)),
        pl.BlockSpec((3 * H, H), lambda l, e, f: (0, 0)),
        pl.BlockSpec((3 * H, H), lambda l, e, f: (0, 0)),
        pl.BlockSpec((1, 3 * H), lambda l, e, f: (0, 0)),
        pl.BlockSpec((1, 3 * H), lambda l, e, f: (0, 0)),
        pl.BlockSpec((1, E, H), lambda l, e, f: (l, 0, 0)),
        pl.BlockSpec((1, 1, E), lambda l, e, f: (l, 0, 0)),
        pl.BlockSpec((1, 1, FB, D), lambda l, e, f: (l, e, f, 0)),
        pl.BlockSpec((1, 1, 1, FB), lambda l, e, f: (l, e, 0, f)),
        pl.BlockSpec((1, 1, D, FB), lambda l, e, f: (l, e, 0, f)),
        pl.BlockSpec((1, 1, 1, D), lambda l, e, f: (l, e, 0, 0)),
    ]
    out = pl.pallas_call(
        _dense_body,
        grid=grid,
        in_specs=specs,
        out_specs=pl.BlockSpec((B, D), lambda l, e, f: (0, 0)),
        out_shape=jax.ShapeDtypeStruct((B, D), jnp.float32),
        scratch_shapes=[
            pltpu.VMEM((B, D), jnp.float32),
            pltpu.VMEM((B, H), jnp.float32),
            pltpu.VMEM((B, E), jnp.float32),
            pltpu.VMEM((B, D), jnp.float32),
            pltpu.VMEM((B, D), jnp.float32),
        ],
        compiler_params=pltpu.CompilerParams(
            dimension_semantics=("arbitrary", "arbitrary", "arbitrary"),
        ),
        interpret=interpret,
    )(x2, Wp, bp3, W_ih, W_hh, b_ih2, b_hh2, Wr, br3, W1e, b1e4, W2e, b2e4)
    return out


def kernel(x, Wp, bp, W_ih, W_hh, b_ih, b_hh, Wr, br, W1e, b1e, W2e, b2e,
           interpret=False):
    batch, seq, d = x.shape
    x2 = x.reshape(batch * seq, d)
    out = _run(x2, Wp, bp.reshape(L, 1, H), W_ih, W_hh,
               b_ih.reshape(1, 3 * H), b_hh.reshape(1, 3 * H),
               Wr, br.reshape(L, 1, E), W1e, b1e.reshape(L, E, 1, F),
               W2e, b2e.reshape(L, E, 1, D), interpret=interpret)
    return out.reshape(batch, seq, d)
